# baseline (speedup 1.0000x reference)
import jax
import jax.numpy as jnp
from jax.experimental import pallas as pl
from jax.experimental.pallas import tpu as pltpu

_VMEM_LIMIT = 56 * 1024 * 1024


def _fused_body(x_ref, w_ref, adjt_ref, adjb_ref, b_ref, o_ref, sup_ref):
    @pl.when(pl.program_id(1) == 0)
    def _():
        sup_ref[...] = jnp.dot(
            x_ref[...].astype(jnp.bfloat16),
            w_ref[...].astype(jnp.bfloat16),
            preferred_element_type=jnp.float32,
        ).astype(jnp.bfloat16)

    h = adjt_ref.shape[0]
    o_ref[:h, :] = (
        jnp.dot(
            adjt_ref[...].astype(jnp.bfloat16),
            sup_ref[...],
            preferred_element_type=jnp.float32,
        )
        + b_ref[...]
    )
    o_ref[h:, :] = (
        jnp.dot(
            adjb_ref[...].astype(jnp.bfloat16),
            sup_ref[...],
            preferred_element_type=jnp.float32,
        )
        + b_ref[...]
    )


def kernel(x, w, adj, bias):
    n, in_f = x.shape
    out_f = w.shape[1]

    x = x.astype(jnp.float32)
    w = w.astype(jnp.float32)
    adj = adj.astype(jnp.float32)
    bias2d = bias.astype(jnp.float32).reshape(1, out_f)

    br = min(n, 512)
    hh = br // 2
    num_tiles = pl.cdiv(n, br)
    num_cores = 2 if num_tiles % 2 == 0 else 1
    tiles_per_core = num_tiles // num_cores

    out = pl.pallas_call(
        _fused_body,
        out_shape=jax.ShapeDtypeStruct((n, out_f), jnp.float32),
        grid=(num_cores, tiles_per_core),
        in_specs=[
            pl.BlockSpec((n, in_f), lambda i, k: (0, 0),
                         pipeline_mode=pl.Buffered(1)),
            pl.BlockSpec((in_f, out_f), lambda i, k: (0, 0),
                         pipeline_mode=pl.Buffered(1)),
            pl.BlockSpec((hh, n),
                         lambda i, k, t=tiles_per_core: (2 * (i * t + k), 0)),
            pl.BlockSpec((hh, n),
                         lambda i, k, t=tiles_per_core: (2 * (i * t + k) + 1, 0)),
            pl.BlockSpec((1, out_f), lambda i, k: (0, 0),
                         pipeline_mode=pl.Buffered(1)),
        ],
        out_specs=pl.BlockSpec((br, out_f),
                               lambda i, k, t=tiles_per_core: (i * t + k, 0)),
        scratch_shapes=[pltpu.VMEM((n, out_f), jnp.bfloat16)],
        compiler_params=pltpu.CompilerParams(
            dimension_semantics=("parallel", "arbitrary"),
            vmem_limit_bytes=_VMEM_LIMIT,
        ),
    )(x, w, adj, adj, bias2d)

    return out
